# Initial kernel scaffold; baseline (speedup 1.0000x reference)
import jax
import jax.numpy as jnp
from jax.experimental import pallas as pl


def kernel(x, adj_t, W1, b1, W2, b2):
    raise NotImplementedError("write your pallas kernel here")


# speedup vs baseline: 19.7414x; 19.7414x over previous
import functools

import jax
import jax.numpy as jnp
from jax import lax
from jax.experimental import pallas as pl
from jax.experimental.pallas import tpu as pltpu
from jax.experimental.pallas import tpu_sc as plsc

N = 10000
E = 320000
D = 128
NC, NS = 2, 16
NW = NC * NS
EPW = E // NW
CH = 80
NCH = EPW // CH
NP = 10240
ZCH = NP // NS
RB = 1000
BM = 2000
GRID = N // BM

_mesh = plsc.VectorSubcoreMesh(
    core_axis_name="c", subcore_axis_name="s", num_cores=NC, num_subcores=NS)


def _deg_body(sd_h, zel_h, degp_h, idxs, ones_v, dega, sem):
    core = lax.axis_index("c")
    sub = lax.axis_index("s")
    wid = core * NS + sub
    for i in range(CH // 16):
        ones_v[pl.ds(i * 16, 16)] = jnp.full((16,), 1.0, jnp.float32)
    pltpu.sync_copy(sd_h.at[wid], idxs)
    pltpu.sync_copy(zel_h, dega.at[pl.ds(sub * ZCH, ZCH)])
    plsc.subcore_barrier()

    def chunk(c, carry):
        pltpu.sync_copy(ones_v, dega.at[idxs.at[c, 1]], add=True)
        return carry

    lax.fori_loop(0, NCH, chunk, 0)
    plsc.subcore_barrier()

    @pl.when(sub < 8)
    def _():
        pltpu.sync_copy(dega.at[pl.ds(sub * 1280, 1280)],
                        degp_h.at[core, 0, pl.ds(sub * 1280, 1280)])


_deg_kernel = pl.kernel(
    _deg_body,
    out_type=jax.ShapeDtypeStruct((NC, 1, NP), jnp.float32),
    mesh=_mesh,
    scratch_types=[
        pltpu.VMEM((NCH, 2, CH), jnp.int32),
        pltpu.VMEM((CH,), jnp.float32),
        pltpu.VMEM_SHARED((NP,), jnp.float32),
        pltpu.SemaphoreType.DMA,
    ],
)


def _agg_body(g_h, sd_h, zrows_h, parts_h, idxs, rows, acc, sem):
    core = lax.axis_index("c")
    sub = lax.axis_index("s")
    wid = core * NS + sub
    pltpu.sync_copy(sd_h.at[wid], idxs)

    @pl.when(sub < 10)
    def _():
        pltpu.sync_copy(zrows_h, acc.at[pl.ds(sub * RB, RB)])

    plsc.subcore_barrier()

    def chunk(c, carry):
        pltpu.async_copy(g_h.at[idxs.at[c, 0]], rows, sem).wait()
        pltpu.sync_copy(rows, acc.at[idxs.at[c, 1]], add=True)
        return carry

    lax.fori_loop(0, NCH, chunk, 0)
    plsc.subcore_barrier()

    @pl.when(sub < 10)
    def _():
        pltpu.sync_copy(acc.at[pl.ds(sub * RB, RB)],
                        parts_h.at[core, pl.ds(sub * RB, RB)])


_agg_kernel = pl.kernel(
    _agg_body,
    out_type=jax.ShapeDtypeStruct((NC, N, D), jnp.float32),
    mesh=_mesh,
    scratch_types=[
        pltpu.VMEM((NCH, 2, CH), jnp.int32),
        pltpu.VMEM((CH, D), jnp.float32),
        pltpu.VMEM_SHARED((N, D), jnp.float32),
        pltpu.SemaphoreType.DMA,
    ],
)


def _dinv_body(dp_ref, dinv_ref):
    d = dp_ref[...]
    dinv_ref[...] = lax.rsqrt(d[0] + d[1] + 1.0)


def _dinv_kernel(dp):
    return pl.pallas_call(
        _dinv_body,
        out_shape=jax.ShapeDtypeStruct((NP // 128, 128), jnp.float32),
    )(dp)


def _mm1_body(x_ref, w_ref, dinv_ref, g_ref):
    h = jnp.dot(x_ref[...], w_ref[...], preferred_element_type=jnp.float32)
    g_ref[...] = dinv_ref[...] * h


def _mm1(x, w1, dinvc):
    return pl.pallas_call(
        _mm1_body,
        grid=(GRID,),
        in_specs=[
            pl.BlockSpec((BM, D), lambda i: (i, 0)),
            pl.BlockSpec((D, D), lambda i: (0, 0)),
            pl.BlockSpec((BM, 1), lambda i: (i, 0)),
        ],
        out_specs=pl.BlockSpec((BM, D), lambda i: (i, 0)),
        out_shape=jax.ShapeDtypeStruct((N, D), jnp.float32),
    )(x, w1, dinvc)


def _mm2_body(p0_ref, p1_ref, g1_ref, dinv_ref, b1_ref, w_ref, g2_ref):
    dinv = dinv_ref[...]
    z = dinv * (p0_ref[...] + p1_ref[...] + g1_ref[...]) + b1_ref[...]
    z = jnp.maximum(z, 0.0)
    h = jnp.dot(z, w_ref[...], preferred_element_type=jnp.float32)
    g2_ref[...] = dinv * h


def _mm2(p0, p1, g1, dinvc, b1r, w2):
    return pl.pallas_call(
        _mm2_body,
        grid=(GRID,),
        in_specs=[
            pl.BlockSpec((BM, D), lambda i: (i, 0)),
            pl.BlockSpec((BM, D), lambda i: (i, 0)),
            pl.BlockSpec((BM, D), lambda i: (i, 0)),
            pl.BlockSpec((BM, 1), lambda i: (i, 0)),
            pl.BlockSpec((1, D), lambda i: (0, 0)),
            pl.BlockSpec((D, D), lambda i: (0, 0)),
        ],
        out_specs=pl.BlockSpec((BM, D), lambda i: (i, 0)),
        out_shape=jax.ShapeDtypeStruct((N, D), jnp.float32),
    )(p0, p1, g1, dinvc, b1r, w2)


def _fin_body(q0_ref, q1_ref, g2_ref, dinv_ref, b2_ref, out_ref):
    out_ref[...] = (dinv_ref[...] * (q0_ref[...] + q1_ref[...] + g2_ref[...])
                    + b2_ref[...])


def _fin(q0, q1, g2, dinvc, b2r):
    return pl.pallas_call(
        _fin_body,
        grid=(GRID,),
        in_specs=[
            pl.BlockSpec((BM, D), lambda i: (i, 0)),
            pl.BlockSpec((BM, D), lambda i: (i, 0)),
            pl.BlockSpec((BM, D), lambda i: (i, 0)),
            pl.BlockSpec((BM, 1), lambda i: (i, 0)),
            pl.BlockSpec((1, D), lambda i: (0, 0)),
        ],
        out_specs=pl.BlockSpec((BM, D), lambda i: (i, 0)),
        out_shape=jax.ShapeDtypeStruct((N, D), jnp.float32),
    )(q0, q1, g2, dinvc, b2r)


def kernel(x, adj_t, W1, b1, W2, b2):
    sd = jnp.transpose(adj_t.reshape(2, NW, NCH, CH), (1, 2, 0, 3))
    zel = jnp.zeros((ZCH,), jnp.float32)
    zrows = jnp.zeros((RB, D), jnp.float32)

    degp = _deg_kernel(sd, zel)
    dp = degp.reshape(NC, NP // 128, 128)
    dinvc = _dinv_kernel(dp).reshape(NP, 1)[:N]
    g1 = _mm1(x, W1, dinvc)
    parts1 = _agg_kernel(g1, sd, zrows)
    g2 = _mm2(parts1[0], parts1[1], g1, dinvc,
              b1.reshape(1, D), W2)
    parts2 = _agg_kernel(g2, sd, zrows)
    return _fin(parts2[0], parts2[1], g2, dinvc,
                b2.reshape(1, D))


# speedup vs baseline: 24.2913x; 1.2305x over previous
import jax
import jax.numpy as jnp
from jax import lax
from jax.experimental import pallas as pl
from jax.experimental.pallas import tpu as pltpu
from jax.experimental.pallas import tpu_sc as plsc

N = 10000
E = 320000
D = 128
NC, NS = 2, 16
NW = NC * NS
EPW = E // NW
CH = 80
NCH = EPW // CH
BLK = 25
NBLK = NCH // BLK
HOFF = 88
NP = 10240
ZCH = NP // NS
RB = 1000
BM = 2000
GRID = N // BM

_mesh = plsc.VectorSubcoreMesh(
    core_axis_name="c", subcore_axis_name="s", num_cores=NC, num_subcores=NS)


def _deg_body(sd_h, ones_h, zel_h, degp_h, idxs, ones_v, dega, isem, dsem):
    core = lax.axis_index("c")
    sub = lax.axis_index("s")
    wid = core * NS + sub
    pltpu.async_copy(sd_h.at[wid], idxs, isem)
    pltpu.sync_copy(ones_h, ones_v)
    pltpu.sync_copy(zel_h, dega.at[pl.ds(sub * ZCH, ZCH)])
    pltpu.make_async_copy(sd_h.at[wid], idxs, isem).wait()
    plsc.subcore_barrier()

    def fire(c, carry):
        pltpu.async_copy(ones_v, dega.at[idxs.at[c, 1]], dsem, add=True)
        return carry

    lax.fori_loop(0, NCH, fire, 0)

    def drain(c, carry):
        pltpu.make_async_copy(ones_v, dega.at[idxs.at[c, 1]], dsem).wait()
        return carry

    lax.fori_loop(0, NCH, drain, 0)
    plsc.subcore_barrier()

    @pl.when(sub < 8)
    def _():
        pltpu.sync_copy(dega.at[pl.ds(sub * 1280, 1280)],
                        degp_h.at[core, 0, pl.ds(sub * 1280, 1280)])


_deg_kernel = pl.kernel(
    _deg_body,
    out_type=jax.ShapeDtypeStruct((NC, 1, NP), jnp.float32),
    mesh=_mesh,
    scratch_types=[
        pltpu.VMEM((NCH, 2, CH), jnp.int32),
        pltpu.VMEM((CH,), jnp.float32),
        pltpu.VMEM_SHARED((NP,), jnp.float32),
        pltpu.SemaphoreType.DMA,
        pltpu.SemaphoreType.DMA,
    ],
)


def _agg_body(g_h, sd_h, zrows_h, parts_h, idxs, rows2, gsem, ssem, acc):
    core = lax.axis_index("c")
    sub = lax.axis_index("s")
    wid = core * NS + sub

    @pl.when(sub < 10)
    def _():
        pltpu.sync_copy(zrows_h, acc.at[pl.ds(sub * RB, RB)])

    plsc.subcore_barrier()

    def halfview(b):
        return rows2.at[pl.ds(pl.multiple_of(b * HOFF, 8), CH)]

    def block(j, carry):
        pltpu.sync_copy(sd_h.at[wid, pl.ds(j * BLK, BLK)], idxs)
        pltpu.async_copy(g_h.at[idxs.at[0, 0]], halfview(0), gsem)

        def step(c, carry2):
            b = lax.rem(c, 2)
            nb = 1 - b
            pltpu.make_async_copy(g_h.at[idxs.at[c, 0]], halfview(b),
                                  gsem).wait()

            @pl.when(c >= 1)
            def _():
                pltpu.make_async_copy(halfview(nb),
                                      acc.at[idxs.at[c - 1, 1]], ssem).wait()

            @pl.when(c < BLK - 1)
            def _():
                pltpu.async_copy(g_h.at[idxs.at[c + 1, 0]], halfview(nb),
                                 gsem)

            pltpu.async_copy(halfview(b), acc.at[idxs.at[c, 1]], ssem,
                             add=True)
            return carry2

        lax.fori_loop(0, BLK, step, 0)
        pltpu.make_async_copy(halfview(lax.rem(BLK - 1, 2)),
                              acc.at[idxs.at[BLK - 1, 1]], ssem).wait()
        return carry

    lax.fori_loop(0, NBLK, block, 0)
    plsc.subcore_barrier()

    @pl.when(sub < 10)
    def _():
        pltpu.sync_copy(acc.at[pl.ds(sub * RB, RB)],
                        parts_h.at[core, pl.ds(sub * RB, RB)])


_agg_kernel = pl.kernel(
    _agg_body,
    out_type=jax.ShapeDtypeStruct((NC, N, D), jnp.float32),
    mesh=_mesh,
    scratch_types=[
        pltpu.VMEM((BLK, 2, CH), jnp.int32),
        pltpu.VMEM((HOFF + CH, D), jnp.float32),
        pltpu.SemaphoreType.DMA,
        pltpu.SemaphoreType.DMA,
        pltpu.VMEM_SHARED((N, D), jnp.float32),
    ],
)


def _mm1_body(x_ref, w_ref, d0_ref, d1_ref, g_ref):
    dinv = lax.rsqrt(d0_ref[...] + d1_ref[...] + 1.0)
    h = jnp.dot(x_ref[...], w_ref[...], preferred_element_type=jnp.float32)
    g_ref[...] = dinv * h


def _mm1(x, w1, d0, d1):
    return pl.pallas_call(
        _mm1_body,
        grid=(GRID,),
        in_specs=[
            pl.BlockSpec((BM, D), lambda i: (i, 0)),
            pl.BlockSpec((D, D), lambda i: (0, 0)),
            pl.BlockSpec((BM, 1), lambda i: (i, 0)),
            pl.BlockSpec((BM, 1), lambda i: (i, 0)),
        ],
        out_specs=pl.BlockSpec((BM, D), lambda i: (i, 0)),
        out_shape=jax.ShapeDtypeStruct((N, D), jnp.float32),
    )(x, w1, d0, d1)


def _mm2_body(p0_ref, p1_ref, g1_ref, d0_ref, d1_ref, b1_ref, w_ref, g2_ref):
    dinv = lax.rsqrt(d0_ref[...] + d1_ref[...] + 1.0)
    z = dinv * (p0_ref[...] + p1_ref[...] + g1_ref[...]) + b1_ref[...]
    z = jnp.maximum(z, 0.0)
    h = jnp.dot(z, w_ref[...], preferred_element_type=jnp.float32)
    g2_ref[...] = dinv * h


def _mm2(p0, p1, g1, d0, d1, b1r, w2):
    return pl.pallas_call(
        _mm2_body,
        grid=(GRID,),
        in_specs=[
            pl.BlockSpec((BM, D), lambda i: (i, 0)),
            pl.BlockSpec((BM, D), lambda i: (i, 0)),
            pl.BlockSpec((BM, D), lambda i: (i, 0)),
            pl.BlockSpec((BM, 1), lambda i: (i, 0)),
            pl.BlockSpec((BM, 1), lambda i: (i, 0)),
            pl.BlockSpec((1, D), lambda i: (0, 0)),
            pl.BlockSpec((D, D), lambda i: (0, 0)),
        ],
        out_specs=pl.BlockSpec((BM, D), lambda i: (i, 0)),
        out_shape=jax.ShapeDtypeStruct((N, D), jnp.float32),
    )(p0, p1, g1, d0, d1, b1r, w2)


def _fin_body(q0_ref, q1_ref, g2_ref, d0_ref, d1_ref, b2_ref, out_ref):
    dinv = lax.rsqrt(d0_ref[...] + d1_ref[...] + 1.0)
    out_ref[...] = (dinv * (q0_ref[...] + q1_ref[...] + g2_ref[...])
                    + b2_ref[...])


def _fin(q0, q1, g2, d0, d1, b2r):
    return pl.pallas_call(
        _fin_body,
        grid=(GRID,),
        in_specs=[
            pl.BlockSpec((BM, D), lambda i: (i, 0)),
            pl.BlockSpec((BM, D), lambda i: (i, 0)),
            pl.BlockSpec((BM, D), lambda i: (i, 0)),
            pl.BlockSpec((BM, 1), lambda i: (i, 0)),
            pl.BlockSpec((BM, 1), lambda i: (i, 0)),
            pl.BlockSpec((1, D), lambda i: (0, 0)),
        ],
        out_specs=pl.BlockSpec((BM, D), lambda i: (i, 0)),
        out_shape=jax.ShapeDtypeStruct((N, D), jnp.float32),
    )(q0, q1, g2, d0, d1, b2r)


def kernel(x, adj_t, W1, b1, W2, b2):
    sd = jnp.transpose(adj_t.reshape(2, NW, NCH, CH), (1, 2, 0, 3))
    ones = jnp.ones((CH,), jnp.float32)
    zel = jnp.zeros((ZCH,), jnp.float32)
    zrows = jnp.zeros((RB, D), jnp.float32)

    degp = _deg_kernel(sd, ones, zel)
    degc = degp.reshape(NC, NP, 1)
    d0, d1 = degc[0, :N], degc[1, :N]
    g1 = _mm1(x, W1, d0, d1)
    parts1 = _agg_kernel(g1, sd, zrows)
    g2 = _mm2(parts1[0], parts1[1], g1, d0, d1,
              b1.reshape(1, D), W2)
    parts2 = _agg_kernel(g2, sd, zrows)
    return _fin(parts2[0], parts2[1], g2, d0, d1,
                b2.reshape(1, D))


# speedup vs baseline: 26.4047x; 1.0870x over previous
import jax
import jax.numpy as jnp
from jax import lax
from jax.experimental import pallas as pl
from jax.experimental.pallas import tpu as pltpu
from jax.experimental.pallas import tpu_sc as plsc

N = 10000
E = 320000
D = 128
NC, NS = 2, 16
NW = NC * NS
EPW = E // NW
CH = 100
NCH = EPW // CH
BLK = 20
NBLK = NCH // BLK
HOFF = 104
NP = 10240
ZCH = NP // NS
RB = 1000
BM = 2000
GRID = N // BM

_mesh = plsc.VectorSubcoreMesh(
    core_axis_name="c", subcore_axis_name="s", num_cores=NC, num_subcores=NS)


def _deg_body(sd_h, ones_h, zel_h, degp_h, idxs, ones_v, dega, isem, dsem):
    core = lax.axis_index("c")
    sub = lax.axis_index("s")
    wid = core * NS + sub
    pltpu.async_copy(sd_h.at[wid], idxs, isem)
    pltpu.sync_copy(ones_h, ones_v)
    pltpu.sync_copy(zel_h, dega.at[pl.ds(sub * ZCH, ZCH)])
    pltpu.make_async_copy(sd_h.at[wid], idxs, isem).wait()
    plsc.subcore_barrier()

    def fire(c, carry):
        pltpu.async_copy(ones_v, dega.at[idxs.at[c, 1]], dsem, add=True)
        return carry

    lax.fori_loop(0, NCH, fire, 0)

    def drain(c, carry):
        pltpu.make_async_copy(ones_v, dega.at[idxs.at[c, 1]], dsem).wait()
        return carry

    lax.fori_loop(0, NCH, drain, 0)
    plsc.subcore_barrier()

    @pl.when(sub < 8)
    def _():
        pltpu.sync_copy(dega.at[pl.ds(sub * 1280, 1280)],
                        degp_h.at[core, 0, pl.ds(sub * 1280, 1280)])


_deg_kernel = pl.kernel(
    _deg_body,
    out_type=jax.ShapeDtypeStruct((NC, 1, NP), jnp.float32),
    mesh=_mesh,
    scratch_types=[
        pltpu.VMEM((NCH, 2, CH), jnp.int32),
        pltpu.VMEM((CH,), jnp.float32),
        pltpu.VMEM_SHARED((NP,), jnp.float32),
        pltpu.SemaphoreType.DMA,
        pltpu.SemaphoreType.DMA,
    ],
)


def _agg_body(g_h, sd_h, zrows_h, parts_h, idxs, rows2, gsem, ssem, acc):
    core = lax.axis_index("c")
    sub = lax.axis_index("s")
    wid = core * NS + sub

    @pl.when(sub < 10)
    def _():
        pltpu.sync_copy(zrows_h, acc.at[pl.ds(sub * RB, RB)])

    plsc.subcore_barrier()

    def halfview(b):
        return rows2.at[pl.ds(pl.multiple_of(b * HOFF, 8), CH)]

    def block(j, carry):
        pltpu.sync_copy(sd_h.at[wid, pl.ds(j * BLK, BLK)], idxs)
        pltpu.async_copy(g_h.at[idxs.at[0, 0]], halfview(0), gsem)

        def step(c, carry2):
            b = lax.rem(c, 2)
            nb = 1 - b
            pltpu.make_async_copy(g_h.at[idxs.at[c, 0]], halfview(b),
                                  gsem).wait()

            @pl.when(c >= 1)
            def _():
                pltpu.make_async_copy(halfview(nb),
                                      acc.at[idxs.at[c - 1, 1]], ssem).wait()

            @pl.when(c < BLK - 1)
            def _():
                pltpu.async_copy(g_h.at[idxs.at[c + 1, 0]], halfview(nb),
                                 gsem)

            pltpu.async_copy(halfview(b), acc.at[idxs.at[c, 1]], ssem,
                             add=True)
            return carry2

        lax.fori_loop(0, BLK, step, 0)
        pltpu.make_async_copy(halfview(lax.rem(BLK - 1, 2)),
                              acc.at[idxs.at[BLK - 1, 1]], ssem).wait()
        return carry

    lax.fori_loop(0, NBLK, block, 0)
    plsc.subcore_barrier()

    @pl.when(sub < 10)
    def _():
        pltpu.sync_copy(acc.at[pl.ds(sub * RB, RB)],
                        parts_h.at[core, pl.ds(sub * RB, RB)])


_agg_kernel = pl.kernel(
    _agg_body,
    out_type=jax.ShapeDtypeStruct((NC, N, D), jnp.float32),
    mesh=_mesh,
    scratch_types=[
        pltpu.VMEM((BLK, 2, CH), jnp.int32),
        pltpu.VMEM((HOFF + CH, D), jnp.float32),
        pltpu.SemaphoreType.DMA,
        pltpu.SemaphoreType.DMA,
        pltpu.VMEM_SHARED((N, D), jnp.float32),
    ],
)


def _mm1_body(x_ref, w_ref, d0_ref, d1_ref, g_ref):
    dinv = lax.rsqrt(d0_ref[...] + d1_ref[...] + 1.0)
    h = jnp.dot(x_ref[...], w_ref[...], preferred_element_type=jnp.float32)
    g_ref[...] = dinv * h


def _mm1(x, w1, d0, d1):
    return pl.pallas_call(
        _mm1_body,
        grid=(GRID,),
        in_specs=[
            pl.BlockSpec((BM, D), lambda i: (i, 0)),
            pl.BlockSpec((D, D), lambda i: (0, 0)),
            pl.BlockSpec((BM, 1), lambda i: (i, 0)),
            pl.BlockSpec((BM, 1), lambda i: (i, 0)),
        ],
        out_specs=pl.BlockSpec((BM, D), lambda i: (i, 0)),
        out_shape=jax.ShapeDtypeStruct((N, D), jnp.float32),
    )(x, w1, d0, d1)


def _mm2_body(p0_ref, p1_ref, g1_ref, d0_ref, d1_ref, b1_ref, w_ref, g2_ref):
    dinv = lax.rsqrt(d0_ref[...] + d1_ref[...] + 1.0)
    z = dinv * (p0_ref[...] + p1_ref[...] + g1_ref[...]) + b1_ref[...]
    z = jnp.maximum(z, 0.0)
    h = jnp.dot(z, w_ref[...], preferred_element_type=jnp.float32)
    g2_ref[...] = dinv * h


def _mm2(p0, p1, g1, d0, d1, b1r, w2):
    return pl.pallas_call(
        _mm2_body,
        grid=(GRID,),
        in_specs=[
            pl.BlockSpec((BM, D), lambda i: (i, 0)),
            pl.BlockSpec((BM, D), lambda i: (i, 0)),
            pl.BlockSpec((BM, D), lambda i: (i, 0)),
            pl.BlockSpec((BM, 1), lambda i: (i, 0)),
            pl.BlockSpec((BM, 1), lambda i: (i, 0)),
            pl.BlockSpec((1, D), lambda i: (0, 0)),
            pl.BlockSpec((D, D), lambda i: (0, 0)),
        ],
        out_specs=pl.BlockSpec((BM, D), lambda i: (i, 0)),
        out_shape=jax.ShapeDtypeStruct((N, D), jnp.float32),
    )(p0, p1, g1, d0, d1, b1r, w2)


def _fin_body(q0_ref, q1_ref, g2_ref, d0_ref, d1_ref, b2_ref, out_ref):
    dinv = lax.rsqrt(d0_ref[...] + d1_ref[...] + 1.0)
    out_ref[...] = (dinv * (q0_ref[...] + q1_ref[...] + g2_ref[...])
                    + b2_ref[...])


def _fin(q0, q1, g2, d0, d1, b2r):
    return pl.pallas_call(
        _fin_body,
        grid=(GRID,),
        in_specs=[
            pl.BlockSpec((BM, D), lambda i: (i, 0)),
            pl.BlockSpec((BM, D), lambda i: (i, 0)),
            pl.BlockSpec((BM, D), lambda i: (i, 0)),
            pl.BlockSpec((BM, 1), lambda i: (i, 0)),
            pl.BlockSpec((BM, 1), lambda i: (i, 0)),
            pl.BlockSpec((1, D), lambda i: (0, 0)),
        ],
        out_specs=pl.BlockSpec((BM, D), lambda i: (i, 0)),
        out_shape=jax.ShapeDtypeStruct((N, D), jnp.float32),
    )(q0, q1, g2, d0, d1, b2r)


def kernel(x, adj_t, W1, b1, W2, b2):
    sd = jnp.transpose(adj_t.reshape(2, NW, NCH, CH), (1, 2, 0, 3))
    ones = jnp.ones((CH,), jnp.float32)
    zel = jnp.zeros((ZCH,), jnp.float32)
    zrows = jnp.zeros((RB, D), jnp.float32)

    degp = _deg_kernel(sd, ones, zel)
    degc = degp.reshape(NC, NP, 1)
    d0, d1 = degc[0, :N], degc[1, :N]
    g1 = _mm1(x, W1, d0, d1)
    parts1 = _agg_kernel(g1, sd, zrows)
    g2 = _mm2(parts1[0], parts1[1], g1, d0, d1,
              b1.reshape(1, D), W2)
    parts2 = _agg_kernel(g2, sd, zrows)
    return _fin(parts2[0], parts2[1], g2, d0, d1,
                b2.reshape(1, D))


# speedup vs baseline: 28.0439x; 1.0621x over previous
import jax
import jax.numpy as jnp
from jax import lax
from jax.experimental import pallas as pl
from jax.experimental.pallas import tpu as pltpu
from jax.experimental.pallas import tpu_sc as plsc

N = 10000
E = 320000
D = 128
NC, NS = 2, 16
NW = NC * NS
EPW = E // NW
CH = 125
NCH = EPW // CH
BLK = 20
NBLK = NCH // BLK
HOFF = 128
NP = 10240
ZCH = NP // NS
RB = 1000
BM = 2000
GRID = N // BM

_mesh = plsc.VectorSubcoreMesh(
    core_axis_name="c", subcore_axis_name="s", num_cores=NC, num_subcores=NS)


def _deg_body(sd_h, ones_h, zel_h, degp_h, idxs, ones_v, dega, isem, dsem):
    core = lax.axis_index("c")
    sub = lax.axis_index("s")
    wid = core * NS + sub
    pltpu.async_copy(sd_h.at[wid], idxs, isem)
    pltpu.sync_copy(ones_h, ones_v)
    pltpu.sync_copy(zel_h, dega.at[pl.ds(sub * ZCH, ZCH)])
    pltpu.make_async_copy(sd_h.at[wid], idxs, isem).wait()
    plsc.subcore_barrier()

    def fire(c, carry):
        pltpu.async_copy(ones_v, dega.at[idxs.at[c, 1]], dsem, add=True)
        return carry

    lax.fori_loop(0, NCH, fire, 0)

    def drain(c, carry):
        pltpu.make_async_copy(ones_v, dega.at[idxs.at[c, 1]], dsem).wait()
        return carry

    lax.fori_loop(0, NCH, drain, 0)
    plsc.subcore_barrier()

    @pl.when(sub < 8)
    def _():
        pltpu.sync_copy(dega.at[pl.ds(sub * 1280, 1280)],
                        degp_h.at[core, 0, pl.ds(sub * 1280, 1280)])


_deg_kernel = pl.kernel(
    _deg_body,
    out_type=jax.ShapeDtypeStruct((NC, 1, NP), jnp.float32),
    mesh=_mesh,
    scratch_types=[
        pltpu.VMEM((NCH, 2, CH), jnp.int32),
        pltpu.VMEM((CH,), jnp.float32),
        pltpu.VMEM_SHARED((NP,), jnp.float32),
        pltpu.SemaphoreType.DMA,
        pltpu.SemaphoreType.DMA,
    ],
)


def _agg_body(g_h, sd_h, zrows_h, parts_h, idxs, rows2, gsem, ssem, acc):
    core = lax.axis_index("c")
    sub = lax.axis_index("s")
    wid = core * NS + sub

    @pl.when(sub < 10)
    def _():
        pltpu.sync_copy(zrows_h, acc.at[pl.ds(sub * RB, RB)])

    plsc.subcore_barrier()

    def halfview(b):
        return rows2.at[pl.ds(pl.multiple_of(b * HOFF, 8), CH)]

    def block(j, carry):
        pltpu.sync_copy(sd_h.at[wid, pl.ds(j * BLK, BLK)], idxs)
        pltpu.async_copy(g_h.at[idxs.at[0, 0]], halfview(0), gsem)

        def step(c, carry2):
            b = lax.rem(c, 2)
            nb = 1 - b
            pltpu.make_async_copy(g_h.at[idxs.at[c, 0]], halfview(b),
                                  gsem).wait()

            @pl.when(c >= 1)
            def _():
                pltpu.make_async_copy(halfview(nb),
                                      acc.at[idxs.at[c - 1, 1]], ssem).wait()

            @pl.when(c < BLK - 1)
            def _():
                pltpu.async_copy(g_h.at[idxs.at[c + 1, 0]], halfview(nb),
                                 gsem)

            pltpu.async_copy(halfview(b), acc.at[idxs.at[c, 1]], ssem,
                             add=True)
            return carry2

        lax.fori_loop(0, BLK, step, 0)
        pltpu.make_async_copy(halfview(lax.rem(BLK - 1, 2)),
                              acc.at[idxs.at[BLK - 1, 1]], ssem).wait()
        return carry

    lax.fori_loop(0, NBLK, block, 0)
    plsc.subcore_barrier()

    @pl.when(sub < 10)
    def _():
        pltpu.sync_copy(acc.at[pl.ds(sub * RB, RB)],
                        parts_h.at[core, pl.ds(sub * RB, RB)])


_agg_kernel = pl.kernel(
    _agg_body,
    out_type=jax.ShapeDtypeStruct((NC, N, D), jnp.float32),
    mesh=_mesh,
    scratch_types=[
        pltpu.VMEM((BLK, 2, CH), jnp.int32),
        pltpu.VMEM((HOFF + CH, D), jnp.float32),
        pltpu.SemaphoreType.DMA,
        pltpu.SemaphoreType.DMA,
        pltpu.VMEM_SHARED((N, D), jnp.float32),
    ],
)


def _mm1_body(x_ref, w_ref, d0_ref, d1_ref, g_ref):
    dinv = lax.rsqrt(d0_ref[...] + d1_ref[...] + 1.0)
    h = jnp.dot(x_ref[...], w_ref[...], preferred_element_type=jnp.float32)
    g_ref[...] = dinv * h


def _mm1(x, w1, d0, d1):
    return pl.pallas_call(
        _mm1_body,
        grid=(GRID,),
        in_specs=[
            pl.BlockSpec((BM, D), lambda i: (i, 0)),
            pl.BlockSpec((D, D), lambda i: (0, 0)),
            pl.BlockSpec((BM, 1), lambda i: (i, 0)),
            pl.BlockSpec((BM, 1), lambda i: (i, 0)),
        ],
        out_specs=pl.BlockSpec((BM, D), lambda i: (i, 0)),
        out_shape=jax.ShapeDtypeStruct((N, D), jnp.float32),
    )(x, w1, d0, d1)


def _mm2_body(p0_ref, p1_ref, g1_ref, d0_ref, d1_ref, b1_ref, w_ref, g2_ref):
    dinv = lax.rsqrt(d0_ref[...] + d1_ref[...] + 1.0)
    z = dinv * (p0_ref[...] + p1_ref[...] + g1_ref[...]) + b1_ref[...]
    z = jnp.maximum(z, 0.0)
    h = jnp.dot(z, w_ref[...], preferred_element_type=jnp.float32)
    g2_ref[...] = dinv * h


def _mm2(p0, p1, g1, d0, d1, b1r, w2):
    return pl.pallas_call(
        _mm2_body,
        grid=(GRID,),
        in_specs=[
            pl.BlockSpec((BM, D), lambda i: (i, 0)),
            pl.BlockSpec((BM, D), lambda i: (i, 0)),
            pl.BlockSpec((BM, D), lambda i: (i, 0)),
            pl.BlockSpec((BM, 1), lambda i: (i, 0)),
            pl.BlockSpec((BM, 1), lambda i: (i, 0)),
            pl.BlockSpec((1, D), lambda i: (0, 0)),
            pl.BlockSpec((D, D), lambda i: (0, 0)),
        ],
        out_specs=pl.BlockSpec((BM, D), lambda i: (i, 0)),
        out_shape=jax.ShapeDtypeStruct((N, D), jnp.float32),
    )(p0, p1, g1, d0, d1, b1r, w2)


def _fin_body(q0_ref, q1_ref, g2_ref, d0_ref, d1_ref, b2_ref, out_ref):
    dinv = lax.rsqrt(d0_ref[...] + d1_ref[...] + 1.0)
    out_ref[...] = (dinv * (q0_ref[...] + q1_ref[...] + g2_ref[...])
                    + b2_ref[...])


def _fin(q0, q1, g2, d0, d1, b2r):
    return pl.pallas_call(
        _fin_body,
        grid=(GRID,),
        in_specs=[
            pl.BlockSpec((BM, D), lambda i: (i, 0)),
            pl.BlockSpec((BM, D), lambda i: (i, 0)),
            pl.BlockSpec((BM, D), lambda i: (i, 0)),
            pl.BlockSpec((BM, 1), lambda i: (i, 0)),
            pl.BlockSpec((BM, 1), lambda i: (i, 0)),
            pl.BlockSpec((1, D), lambda i: (0, 0)),
        ],
        out_specs=pl.BlockSpec((BM, D), lambda i: (i, 0)),
        out_shape=jax.ShapeDtypeStruct((N, D), jnp.float32),
    )(q0, q1, g2, d0, d1, b2r)


def kernel(x, adj_t, W1, b1, W2, b2):
    sd = jnp.transpose(adj_t.reshape(2, NW, NCH, CH), (1, 2, 0, 3))
    ones = jnp.ones((CH,), jnp.float32)
    zel = jnp.zeros((ZCH,), jnp.float32)
    zrows = jnp.zeros((RB, D), jnp.float32)

    degp = _deg_kernel(sd, ones, zel)
    degc = degp.reshape(NC, NP, 1)
    d0, d1 = degc[0, :N], degc[1, :N]
    g1 = _mm1(x, W1, d0, d1)
    parts1 = _agg_kernel(g1, sd, zrows)
    g2 = _mm2(parts1[0], parts1[1], g1, d0, d1,
              b1.reshape(1, D), W2)
    parts2 = _agg_kernel(g2, sd, zrows)
    return _fin(parts2[0], parts2[1], g2, d0, d1,
                b2.reshape(1, D))
